# initial kernel scaffold (unmeasured)
import jax
import jax.numpy as jnp
from jax import lax
from jax.experimental import pallas as pl
from jax.experimental.pallas import tpu as pltpu

N_DEV = 4


def kernel(x, w_mat):
    m_per, k = x.shape
    _, n_per = w_mat.shape

    def body(x_ref, w_ref, out_ref, comm_ref, send_sems, recv_sems):
        my_pos = lax.axis_index("i")
        left = (my_pos - 1) % N_DEV
        right = (my_pos + 1) % N_DEV

        barrier_sem = pltpu.get_barrier_semaphore()
        for nbr in [left, right]:
            pl.semaphore_signal(
                barrier_sem, inc=1,
                device_id=(nbr,), device_id_type=pl.DeviceIdType.MESH,
            )
        pl.semaphore_wait(barrier_sem, 2)

        comm_ref[0, :, :] = x_ref[:, :]
        out_ref[pl.ds(my_pos * m_per, m_per), :] = jnp.maximum(
            jnp.dot(x_ref[:, :], w_ref[:, :], preferred_element_type=jnp.float32),
            0.0,
        )

        for h in range(N_DEV - 1):
            send_slot = h % 2
            recv_slot = (h + 1) % 2
            rdma = pltpu.make_async_remote_copy(
                src_ref=comm_ref.at[send_slot],
                dst_ref=comm_ref.at[recv_slot],
                send_sem=send_sems.at[send_slot],
                recv_sem=recv_sems.at[recv_slot],
                device_id=(right,),
                device_id_type=pl.DeviceIdType.MESH,
            )
            rdma.start()
            rdma.wait()

            origin = (my_pos - h - 1) % N_DEV
            out_ref[pl.ds(origin * m_per, m_per), :] = jnp.maximum(
                jnp.dot(
                    comm_ref[recv_slot, :, :],
                    w_ref[:, :],
                    preferred_element_type=jnp.float32,
                ),
                0.0,
            )

    return pl.pallas_call(
        body,
        out_shape=jax.ShapeDtypeStruct((N_DEV * m_per, n_per), jnp.float32),
        in_specs=[
            pl.BlockSpec(memory_space=pltpu.VMEM),
            pl.BlockSpec(memory_space=pltpu.VMEM),
        ],
        out_specs=pl.BlockSpec(memory_space=pltpu.VMEM),
        scratch_shapes=[
            pltpu.VMEM((2, m_per, k), x.dtype),
            pltpu.SemaphoreType.DMA((2,)),
            pltpu.SemaphoreType.DMA((2,)),
        ],
        compiler_params=pltpu.CompilerParams(collective_id=0),
    )(x, w_mat)


# baseline (device time: 423379 ns/iter reference)
import jax
import jax.numpy as jnp
from jax import lax
from jax.experimental import pallas as pl
from jax.experimental.pallas import tpu as pltpu

N_DEV = 4


def kernel(x, w_mat):
    m_per, k = x.shape
    _, n_per = w_mat.shape

    xb = x.astype(jnp.bfloat16)
    wb = w_mat.astype(jnp.bfloat16)

    def body(x_ref, w_ref, out_ref, comm_ref, stage_ref,
             send_sems, recv_sems, copy_sems):
        my_pos = lax.axis_index("i")
        left = (my_pos - 1) % N_DEV
        right = (my_pos + 1) % N_DEV

        barrier_sem = pltpu.get_barrier_semaphore()
        for nbr in [left, right]:
            pl.semaphore_signal(
                barrier_sem, inc=1,
                device_id=(nbr,), device_id_type=pl.DeviceIdType.MESH,
            )
        pl.semaphore_wait(barrier_sem, 2)

        def store_block(slot, origin):
            copy = pltpu.make_async_copy(
                stage_ref.at[slot],
                out_ref.at[pl.ds(origin * m_per, m_per), :],
                copy_sems.at[slot],
            )
            copy.start()
            copy.wait()

        comm_ref[0, :, :] = x_ref[:, :]
        stage_ref[0, :, :] = jnp.maximum(
            jnp.dot(x_ref[:, :], w_ref[:, :], preferred_element_type=jnp.float32),
            0.0,
        )
        store_block(0, my_pos)

        for h in range(N_DEV - 1):
            send_slot = h % 2
            recv_slot = (h + 1) % 2
            rdma = pltpu.make_async_remote_copy(
                src_ref=comm_ref.at[send_slot],
                dst_ref=comm_ref.at[recv_slot],
                send_sem=send_sems.at[send_slot],
                recv_sem=recv_sems.at[recv_slot],
                device_id=(right,),
                device_id_type=pl.DeviceIdType.MESH,
            )
            rdma.start()
            rdma.wait()

            origin = (my_pos - h - 1) % N_DEV
            stage_ref[0, :, :] = jnp.maximum(
                jnp.dot(
                    comm_ref[recv_slot, :, :],
                    w_ref[:, :],
                    preferred_element_type=jnp.float32,
                ),
                0.0,
            )
            store_block(0, origin)

    return pl.pallas_call(
        body,
        out_shape=jax.ShapeDtypeStruct((N_DEV * m_per, n_per), jnp.float32),
        in_specs=[
            pl.BlockSpec(memory_space=pltpu.VMEM),
            pl.BlockSpec(memory_space=pltpu.VMEM),
        ],
        out_specs=pl.BlockSpec(memory_space=pl.ANY),
        scratch_shapes=[
            pltpu.VMEM((2, m_per, k), jnp.bfloat16),
            pltpu.VMEM((1, m_per, n_per), jnp.float32),
            pltpu.SemaphoreType.DMA((2,)),
            pltpu.SemaphoreType.DMA((2,)),
            pltpu.SemaphoreType.DMA((2,)),
        ],
        compiler_params=pltpu.CompilerParams(
            collective_id=0,
            vmem_limit_bytes=60 * 1024 * 1024,
        ),
    )(xb, wb)


# device time: 223368 ns/iter; 1.8954x vs baseline; 1.8954x over previous
import functools

import jax
import jax.numpy as jnp
from jax import lax
from jax.experimental import pallas as pl
from jax.experimental.pallas import tpu as pltpu

N_DEV = 4


def kernel(x, w_mat):
    m_per, k = x.shape
    _, n_per = w_mat.shape
    half = m_per // 2

    xb = x.astype(jnp.bfloat16)
    wb = w_mat.astype(jnp.bfloat16)

    def body(x_hbm, w_ref, out_ref, top_ref, bot_ref, stage_ref,
             load_sems, sendR_sems, recvR_sems, sendL_sems, recvL_sems,
             copy_sems):
        my = lax.axis_index("i")
        left = (my - 1) % N_DEV
        right = (my + 1) % N_DEV

        ld_top = pltpu.make_async_copy(
            x_hbm.at[pl.ds(0, half), :], top_ref.at[0], load_sems.at[0])
        ld_bot = pltpu.make_async_copy(
            x_hbm.at[pl.ds(half, half), :], bot_ref.at[0], load_sems.at[1])
        ld_top.start()
        ld_bot.start()

        barrier_sem = pltpu.get_barrier_semaphore()
        for nbr in [left, right]:
            pl.semaphore_signal(
                barrier_sem, inc=1,
                device_id=(nbr,), device_id_type=pl.DeviceIdType.MESH,
            )
        pl.semaphore_wait(barrier_sem, 2)

        ld_top.wait()
        ld_bot.wait()

        pending = [None, None]
        counter = [0]

        def do_block(buf_ref, s, row_start):
            slot = counter[0] % 2
            counter[0] += 1
            if pending[slot] is not None:
                pending[slot].wait()
            stage_ref[slot, :, :] = jnp.maximum(
                jnp.dot(buf_ref[s, :, :], w_ref[:, :],
                        preferred_element_type=jnp.float32),
                0.0,
            )
            cp = pltpu.make_async_copy(
                stage_ref.at[slot],
                out_ref.at[pl.ds(row_start, half), :],
                copy_sems.at[slot],
            )
            cp.start()
            pending[slot] = cp

        sends = []
        for h in range(N_DEV - 1):
            rR = pltpu.make_async_remote_copy(
                src_ref=top_ref.at[h],
                dst_ref=top_ref.at[h + 1],
                send_sem=sendR_sems.at[h],
                recv_sem=recvR_sems.at[h],
                device_id=(right,),
                device_id_type=pl.DeviceIdType.MESH,
            )
            rL = pltpu.make_async_remote_copy(
                src_ref=bot_ref.at[h],
                dst_ref=bot_ref.at[h + 1],
                send_sem=sendL_sems.at[h],
                recv_sem=recvL_sems.at[h],
                device_id=(left,),
                device_id_type=pl.DeviceIdType.MESH,
            )
            rR.start()
            rL.start()
            sends += [rR, rL]

            do_block(top_ref, h, ((my - h) % N_DEV) * m_per)
            do_block(bot_ref, h, ((my + h) % N_DEV) * m_per + half)

            rR.wait_recv()
            rL.wait_recv()

        s = N_DEV - 1
        do_block(top_ref, s, ((my - s) % N_DEV) * m_per)
        do_block(bot_ref, s, ((my + s) % N_DEV) * m_per + half)

        for p in pending:
            if p is not None:
                p.wait()
        for snd in sends:
            snd.wait_send()

        @functools.partial(
            pl.run_scoped, second_barrier=pltpu.SemaphoreType.REGULAR)
        def _(second_barrier):
            for nbr in [left, right]:
                pl.semaphore_signal(
                    second_barrier, inc=1,
                    device_id=(nbr,), device_id_type=pl.DeviceIdType.MESH,
                )
            pl.semaphore_wait(second_barrier, 2)

    return pl.pallas_call(
        body,
        out_shape=jax.ShapeDtypeStruct((N_DEV * m_per, n_per), jnp.float32),
        in_specs=[
            pl.BlockSpec(memory_space=pl.ANY),
            pl.BlockSpec(memory_space=pltpu.VMEM),
        ],
        out_specs=pl.BlockSpec(memory_space=pl.ANY),
        scratch_shapes=[
            pltpu.VMEM((N_DEV, half, k), jnp.bfloat16),
            pltpu.VMEM((N_DEV, half, k), jnp.bfloat16),
            pltpu.VMEM((2, half, n_per), jnp.float32),
            pltpu.SemaphoreType.DMA((2,)),
            pltpu.SemaphoreType.DMA((N_DEV - 1,)),
            pltpu.SemaphoreType.DMA((N_DEV - 1,)),
            pltpu.SemaphoreType.DMA((N_DEV - 1,)),
            pltpu.SemaphoreType.DMA((N_DEV - 1,)),
            pltpu.SemaphoreType.DMA((2,)),
        ],
        compiler_params=pltpu.CompilerParams(
            collective_id=0,
            vmem_limit_bytes=62 * 1024 * 1024,
        ),
    )(xb, wb)


# device time: 199933 ns/iter; 2.1176x vs baseline; 1.1172x over previous
import functools

import jax
import jax.numpy as jnp
from jax import lax
from jax.experimental import pallas as pl
from jax.experimental.pallas import tpu as pltpu

N_DEV = 4
N_WTILES = 16


def kernel(x, w_mat):
    m_per, k = x.shape
    _, n_per = w_mat.shape
    half = m_per // 2

    xb = x.astype(jnp.bfloat16)

    def body(x_hbm, w_hbm, out_ref, top_ref, bot_ref, wb_ref, wtmp_ref,
             stage_ref, load_sems, wload_sem, sendR_sems, recvR_sems,
             sendL_sems, recvL_sems, copy_sems):
        my = lax.axis_index("i")
        left = (my - 1) % N_DEV
        right = (my + 1) % N_DEV
        kt = k // N_WTILES

        ld_top = pltpu.make_async_copy(
            x_hbm.at[pl.ds(0, half), :], top_ref.at[0], load_sems.at[0])
        ld_bot = pltpu.make_async_copy(
            x_hbm.at[pl.ds(half, half), :], bot_ref.at[0], load_sems.at[1])
        ld_top.start()
        ld_bot.start()

        def w_tile_copy(t):
            return pltpu.make_async_copy(
                w_hbm.at[pl.ds(t * kt, kt), :], wtmp_ref, wload_sem)

        wdma = w_tile_copy(0)
        wdma.start()

        barrier_sem = pltpu.get_barrier_semaphore()
        for nbr in [left, right]:
            pl.semaphore_signal(
                barrier_sem, inc=1,
                device_id=(nbr,), device_id_type=pl.DeviceIdType.MESH,
            )
        pl.semaphore_wait(barrier_sem, 2)

        ld_top.wait()
        ld_bot.wait()

        pending = [None, None]
        counter = [0]

        nh = n_per // 2

        def do_block(buf_ref, s, row_start):
            for j in range(2):
                slot = counter[0] % 2
                counter[0] += 1
                if pending[slot] is not None:
                    pending[slot].wait()
                stage_ref[slot, :, :] = jnp.maximum(
                    jnp.dot(buf_ref[s, :, :],
                            wb_ref[:, pl.ds(j * nh, nh)],
                            preferred_element_type=jnp.float32),
                    0.0,
                )
                cp = pltpu.make_async_copy(
                    stage_ref.at[slot],
                    out_ref.at[pl.ds(row_start, half), pl.ds(j * nh, nh)],
                    copy_sems.at[slot],
                )
                cp.start()
                pending[slot] = cp

        def top_row(h):
            return ((my - h) % N_DEV) * m_per

        def bot_row(h):
            return ((my + h) % N_DEV) * m_per + half

        sends = []

        def start_hop(h):
            s, d = h % 3, (h + 1) % 3
            rR = pltpu.make_async_remote_copy(
                src_ref=top_ref.at[s], dst_ref=top_ref.at[d],
                send_sem=sendR_sems.at[h], recv_sem=recvR_sems.at[h],
                device_id=(right,), device_id_type=pl.DeviceIdType.MESH,
            )
            rL = pltpu.make_async_remote_copy(
                src_ref=bot_ref.at[s], dst_ref=bot_ref.at[d],
                send_sem=sendL_sems.at[h], recv_sem=recvL_sems.at[h],
                device_id=(left,), device_id_type=pl.DeviceIdType.MESH,
            )
            rR.start()
            rL.start()
            sends.extend([rR, rL])
            return rR, rL

        rR, rL = start_hop(0)
        for t in range(N_WTILES):
            wdma.wait()
            wb_ref[pl.ds(t * kt, kt), :] = wtmp_ref[:, :].astype(jnp.bfloat16)
            if t + 1 < N_WTILES:
                wdma = w_tile_copy(t + 1)
                wdma.start()
        do_block(top_ref, 0, top_row(0))
        rR.wait_recv()
        rL.wait_recv()

        rR, rL = start_hop(1)
        do_block(bot_ref, 0, bot_row(0))
        do_block(top_ref, 1, top_row(1))
        do_block(bot_ref, 1, bot_row(1))
        rR.wait_recv()
        rL.wait_recv()

        rR, rL = start_hop(2)
        do_block(top_ref, 2, top_row(2))
        do_block(bot_ref, 2, bot_row(2))
        rR.wait_recv()
        rL.wait_recv()

        do_block(top_ref, 0, top_row(3))
        do_block(bot_ref, 0, bot_row(3))

        for p in pending:
            if p is not None:
                p.wait()
        for snd in sends:
            snd.wait_send()

        @functools.partial(
            pl.run_scoped, second_barrier=pltpu.SemaphoreType.REGULAR)
        def _(second_barrier):
            for nbr in [left, right]:
                pl.semaphore_signal(
                    second_barrier, inc=1,
                    device_id=(nbr,), device_id_type=pl.DeviceIdType.MESH,
                )
            pl.semaphore_wait(second_barrier, 2)

    return pl.pallas_call(
        body,
        out_shape=jax.ShapeDtypeStruct((N_DEV * m_per, n_per), jnp.float32),
        in_specs=[
            pl.BlockSpec(memory_space=pl.ANY),
            pl.BlockSpec(memory_space=pl.ANY),
        ],
        out_specs=pl.BlockSpec(memory_space=pl.ANY),
        scratch_shapes=[
            pltpu.VMEM((3, half, k), jnp.bfloat16),
            pltpu.VMEM((3, half, k), jnp.bfloat16),
            pltpu.VMEM((k, n_per), jnp.bfloat16),
            pltpu.VMEM((k // N_WTILES, n_per), jnp.float32),
            pltpu.VMEM((2, half, n_per // 2), jnp.float32),
            pltpu.SemaphoreType.DMA((2,)),
            pltpu.SemaphoreType.DMA,
            pltpu.SemaphoreType.DMA((N_DEV - 1,)),
            pltpu.SemaphoreType.DMA((N_DEV - 1,)),
            pltpu.SemaphoreType.DMA((N_DEV - 1,)),
            pltpu.SemaphoreType.DMA((N_DEV - 1,)),
            pltpu.SemaphoreType.DMA((2,)),
        ],
        compiler_params=pltpu.CompilerParams(
            collective_id=0,
            vmem_limit_bytes=62 * 1024 * 1024,
        ),
    )(xb, w_mat)
